# baseline (speedup 1.0000x reference)
import functools

import jax
import jax.numpy as jnp
from jax import lax
from jax.experimental import pallas as pl
from jax.experimental.pallas import tpu as pltpu
from jax.experimental.pallas import tpu_sc as plsc

ALPHA = 0.2
EPS = 1e-16
ECLAMP = 50.0
F32 = jnp.float32
I32 = jnp.int32


def _tc_node_matmul(x, W, a, N, D_IN, D_OUT):
    BN = 1000 if N % 1000 == 0 else N

    def body(x_ref, W_ref, a_ref, h1_ref, h2_ref, s1_ref, s2_ref):
        xb = x_ref[...]
        h1 = jnp.dot(xb, W_ref[0:D_IN, :], preferred_element_type=F32)
        h2 = jnp.dot(xb, W_ref[D_IN:2 * D_IN, :], preferred_element_type=F32)
        h1_ref[...] = h1
        h2_ref[...] = h2
        av = a_ref[...]
        s1_ref[...] = jnp.dot(h1, av, preferred_element_type=F32)
        s2_ref[...] = jnp.dot(h2, av, preferred_element_type=F32)

    DW = W.shape[0]
    return pl.pallas_call(
        body,
        grid=(N // BN,),
        in_specs=[
            pl.BlockSpec((BN, D_IN), lambda i: (i, 0)),
            pl.BlockSpec((DW, D_OUT), lambda i: (0, 0)),
            pl.BlockSpec((D_OUT, 1), lambda i: (0, 0)),
        ],
        out_specs=[
            pl.BlockSpec((BN, D_OUT), lambda i: (i, 0)),
            pl.BlockSpec((BN, D_OUT), lambda i: (i, 0)),
            pl.BlockSpec((BN, 1), lambda i: (i, 0)),
            pl.BlockSpec((BN, 1), lambda i: (i, 0)),
        ],
        out_shape=[
            jax.ShapeDtypeStruct((N, D_OUT), F32),
            jax.ShapeDtypeStruct((N, D_OUT), F32),
            jax.ShapeDtypeStruct((N, 1), F32),
            jax.ShapeDtypeStruct((N, 1), F32),
        ],
    )(x, W, a)


def _tc_edge_maps(ett_p, W, a, E_pad, D_IN, D_REL, D_OUT):
    BE = 8192 if E_pad % 8192 == 0 else E_pad
    DW = W.shape[0]

    def body(ett_ref, W_ref, a_ref, z_ref, s3_ref):
        z = jnp.dot(ett_ref[...], W_ref[2 * D_IN:, :], preferred_element_type=F32)
        z_ref[...] = z
        s3_ref[...] = jnp.dot(z, a_ref[...], preferred_element_type=F32)

    return pl.pallas_call(
        body,
        grid=(E_pad // BE,),
        in_specs=[
            pl.BlockSpec((BE, D_REL), lambda i: (i, 0)),
            pl.BlockSpec((DW, D_OUT), lambda i: (0, 0)),
            pl.BlockSpec((D_OUT, 1), lambda i: (0, 0)),
        ],
        out_specs=[
            pl.BlockSpec((BE, D_OUT), lambda i: (i, 0)),
            pl.BlockSpec((BE, 1), lambda i: (i, 0)),
        ],
        out_shape=[
            jax.ShapeDtypeStruct((E_pad, D_OUT), F32),
            jax.ShapeDtypeStruct((E_pad, 1), F32),
        ],
    )(ett_p, W, a)


def _tc_finalize(h1, denp, acc, N, D_OUT, NC):
    BN = 1000 if N % 1000 == 0 else N

    def body(h1_ref, dp_ref, acc_ref, o_ref):
        den = dp_ref[0] + dp_ref[1]
        r = 1.0 / (den + EPS)
        t = (h1_ref[...] * den + acc_ref[0] + acc_ref[1]) * r
        o_ref[...] = jnp.where(t > 0.0, t, jnp.exp(t) - 1.0)

    return pl.pallas_call(
        body,
        grid=(N // BN,),
        in_specs=[
            pl.BlockSpec((BN, D_OUT), lambda i: (i, 0)),
            pl.BlockSpec((NC, BN, 1), lambda i: (0, i, 0)),
            pl.BlockSpec((NC, BN, D_OUT), lambda i: (0, i, 0)),
        ],
        out_specs=pl.BlockSpec((BN, D_OUT), lambda i: (i, 0)),
        out_shape=jax.ShapeDtypeStruct((N, D_OUT), F32),
    )(h1, denp, acc)


def kernel(edge_index, x, edge_type_embed, W, a):
    N, D_IN = x.shape
    E = edge_index.shape[1]
    D_REL = edge_type_embed.shape[1]
    D_OUT = W.shape[1]

    info = plsc.get_sparse_core_info()
    NC, NS, L = info.num_cores, info.num_subcores, info.num_lanes
    NW = NC * NS

    CHUNK = 2048
    GROUPS = CHUNK // L
    PT = ((E + NW * CHUNK - 1) // (NW * CHUNK)) * CHUNK
    E_pad = PT * NW
    n_chunks = PT // CHUNK
    NB = PT // 128
    NSTR = (N + 1 + 127) // 128
    NT = NSTR * 128
    KSTR = (NSTR + NS - 1) // NS

    mesh = plsc.VectorSubcoreMesh(core_axis_name="c", subcore_axis_name="s")

    src = jnp.concatenate([edge_index[0], jnp.full((E_pad - E,), N, I32)])
    dst = jnp.concatenate([edge_index[1], jnp.zeros((E_pad - E,), I32)])
    ett_p = jnp.pad(edge_type_embed, ((0, E_pad - E), (0, 0)))

    h1, h2, s1, s2 = _tc_node_matmul(x, W, a, N, D_IN, D_OUT)
    z, s3 = _tc_edge_maps(ett_p, W, a, E_pad, D_IN, D_REL, D_OUT)
    s1p = jnp.pad(s1[:, 0], (0, NT - N))
    s2p = jnp.pad(s2[:, 0], (0, NT - N))
    s3p = s3[:, 0]

    CH0 = (n_chunks * 2) // 3
    CH1 = n_chunks * 2 - CH0
    PT0 = CH0 * CHUNK
    PT1 = CH1 * CHUNK

    def _tile_span(c, s):
        base = jnp.where(c == 0, s * PT0, NS * PT0 + s * PT1)
        nch = jnp.where(c == 0, CH0, CH1)
        return base, nch

    @functools.partial(
        pl.kernel,
        out_type=(jax.ShapeDtypeStruct((E_pad,), F32),
                  jax.ShapeDtypeStruct((NC, NT), F32)),
        mesh=mesh,
        compiler_params=pltpu.CompilerParams(needs_layout_passes=False),
        scratch_types=[
            pltpu.VMEM((NT,), F32),
            pltpu.VMEM((NT,), F32),
            pltpu.VMEM((CHUNK,), I32),
            pltpu.VMEM((CHUNK,), I32),
            pltpu.VMEM((CHUNK,), F32),
            pltpu.VMEM((CHUNK,), F32),
            pltpu.VMEM((128,), I32),
            pltpu.VMEM((128,), F32),
            pltpu.VMEM_SHARED((NT,), F32),
        ],
    )
    def sc_front(src_h, dst_h, s3_h, s1_h, s2_h,
                 p_h, den_h,
                 s1_t, s2_t, srcb, dstb, s3b, pb, isrc, zbuf, den_sp):
        c = lax.axis_index("c")
        s = lax.axis_index("s")
        base, nch = _tile_span(c, s)
        pltpu.sync_copy(s1_h, s1_t)
        pltpu.sync_copy(s2_h, s2_t)

        def z1b(g, _):
            zbuf[pl.ds(g * L, L)] = jnp.zeros((L,), F32)
            return 0

        lax.fori_loop(0, 128 // L, z1b, 0)

        def z2(k, _):
            j = k * NS + s

            @pl.when(j < NSTR)
            def _():
                pltpu.sync_copy(zbuf, den_sp.at[pl.ds(j * 128, 128)])
            return 0

        lax.fori_loop(0, KSTR, z2, 0)
        plsc.subcore_barrier()

        def chunk_body(k, _):
            off = base + k * CHUNK
            pltpu.sync_copy(src_h.at[pl.ds(off, CHUNK)], srcb)
            pltpu.sync_copy(dst_h.at[pl.ds(off, CHUNK)], dstb)
            pltpu.sync_copy(s3_h.at[pl.ds(off, CHUNK)], s3b)

            def g_body(g, _):
                sl = pl.ds(g * L, L)
                e = (plsc.load_gather(s1_t, [srcb[sl]])
                     + plsc.load_gather(s2_t, [dstb[sl]])
                     + s3b[sl])
                e = jnp.where(e > 0.0, e, ALPHA * e)
                pb[sl] = jnp.exp(jnp.minimum(e, ECLAMP))
                return 0

            lax.fori_loop(0, GROUPS, g_body, 0)
            pltpu.sync_copy(pb, p_h.at[pl.ds(off, CHUNK)])

            def blk(j, _):
                def cp(g2, _):
                    isrc[pl.ds(g2 * L, L)] = srcb[pl.ds(j * 128 + g2 * L, L)]
                    return 0

                lax.fori_loop(0, 128 // L, cp, 0)
                pltpu.sync_copy(pb.at[pl.ds(j * 128, 128)], den_sp.at[isrc],
                                add=True)
                return 0

            lax.fori_loop(0, CHUNK // 128, blk, 0)
            return 0

        lax.fori_loop(0, nch, chunk_body, 0)
        plsc.subcore_barrier()

        def outs(k, _):
            j = k * NS + s

            @pl.when(j < NSTR)
            def _():
                pltpu.sync_copy(den_sp.at[pl.ds(j * 128, 128)],
                                den_h.at[c, pl.ds(j * 128, 128)])
            return 0

        lax.fori_loop(0, KSTR, outs, 0)

    p, denp = sc_front(src, dst, s3p, s1p, s2p)

    @functools.partial(
        pl.kernel,
        out_type=jax.ShapeDtypeStruct((NC, NT, D_OUT), F32),
        mesh=mesh,
        compiler_params=pltpu.CompilerParams(needs_layout_passes=False),
        scratch_types=[
            pltpu.VMEM((CHUNK,), I32),
            pltpu.VMEM((CHUNK,), I32),
            pltpu.VMEM((CHUNK,), F32),
            pltpu.VMEM((128,), I32),
            pltpu.VMEM((128,), I32),
            pltpu.VMEM((128, 128), F32),
            pltpu.VMEM((128, 128), F32),
            pltpu.SemaphoreType.DMA,
            pltpu.VMEM_SHARED((NT, 128), F32),
        ],
    )
    def sc_acc(src_h, dst_h, p_h, h2_h, z_h, acc_h,
               srcb, dstb, pb, isrc, idst, rows, zrows, sem, acc_sp):
        c = lax.axis_index("c")
        s = lax.axis_index("s")
        base, nch = _tile_span(c, s)

        def z1(i, _):
            for g in range(128 // L):
                rows[i, pl.ds(g * L, L)] = jnp.zeros((L,), F32)
            return 0

        lax.fori_loop(0, 128, z1, 0)

        def z2(k, _):
            j = k * NS + s

            @pl.when(j < NSTR)
            def _():
                pltpu.sync_copy(rows, acc_sp.at[pl.ds(j * 128, 128), :])
            return 0

        lax.fori_loop(0, KSTR, z2, 0)
        plsc.subcore_barrier()

        def chunk_body(k, _):
            off = base + k * CHUNK
            pltpu.sync_copy(src_h.at[pl.ds(off, CHUNK)], srcb)
            pltpu.sync_copy(dst_h.at[pl.ds(off, CHUNK)], dstb)
            pltpu.sync_copy(p_h.at[pl.ds(off, CHUNK)], pb)

            def blk(j, _):
                def cp(g2, _):
                    sl = pl.ds(g2 * L, L)
                    bsl = pl.ds(j * 128 + g2 * L, L)
                    isrc[sl] = srcb[bsl]
                    idst[sl] = dstb[bsl]
                    return 0

                lax.fori_loop(0, 128 // L, cp, 0)
                gth = pltpu.async_copy(h2_h.at[idst], rows, sem)
                pltpu.sync_copy(z_h.at[pl.ds(off + j * 128, 128), :], zrows)
                gth.wait()

                def se(e, _):
                    idxv = jnp.full((L,), 0, I32) + (j * 128 + e)
                    pv = plsc.load_gather(pb, [idxv])
                    for g in range(128 // L):
                        sl2 = pl.ds(g * L, L)
                        rows[e, sl2] = (rows[e, sl2] + zrows[e, sl2]) * pv
                    return 0

                lax.fori_loop(0, 128, se, 0)
                pltpu.sync_copy(rows, acc_sp.at[isrc], add=True)
                return 0

            lax.fori_loop(0, CHUNK // 128, blk, 0)
            return 0

        lax.fori_loop(0, nch, chunk_body, 0)
        plsc.subcore_barrier()

        def outs(k, _):
            j = k * NS + s

            @pl.when(j < NSTR)
            def _():
                pltpu.sync_copy(acc_sp.at[pl.ds(j * 128, 128), :],
                                acc_h.at[c, pl.ds(j * 128, 128), :])
            return 0

        lax.fori_loop(0, KSTR, outs, 0)

    acc = sc_acc(src, dst, p, h2, z)

    return _tc_finalize(h1, denp.reshape(NC, NT, 1)[:, :N], acc[:, :N],
                        N, D_OUT, NC)


# speedup vs baseline: 1.2398x; 1.2398x over previous
import functools

import jax
import jax.numpy as jnp
from jax import lax
from jax.experimental import pallas as pl
from jax.experimental.pallas import tpu as pltpu
from jax.experimental.pallas import tpu_sc as plsc

ALPHA = 0.2
EPS = 1e-16
ECLAMP = 50.0
F32 = jnp.float32
I32 = jnp.int32


def _tc_node_matmul(x, W, a, N, D_IN, D_OUT):
    BN = 1000 if N % 1000 == 0 else N

    def body(x_ref, W_ref, a_ref, h1_ref, h2_ref, s1_ref, s2_ref):
        xb = x_ref[...]
        h1 = jnp.dot(xb, W_ref[0:D_IN, :], preferred_element_type=F32)
        h2 = jnp.dot(xb, W_ref[D_IN:2 * D_IN, :], preferred_element_type=F32)
        h1_ref[...] = h1
        h2_ref[...] = h2
        av = a_ref[...]
        s1_ref[...] = jnp.dot(h1, av, preferred_element_type=F32)
        s2_ref[...] = jnp.dot(h2, av, preferred_element_type=F32)

    DW = W.shape[0]
    return pl.pallas_call(
        body,
        grid=(N // BN,),
        in_specs=[
            pl.BlockSpec((BN, D_IN), lambda i: (i, 0)),
            pl.BlockSpec((DW, D_OUT), lambda i: (0, 0)),
            pl.BlockSpec((D_OUT, 1), lambda i: (0, 0)),
        ],
        out_specs=[
            pl.BlockSpec((BN, D_OUT), lambda i: (i, 0)),
            pl.BlockSpec((BN, D_OUT), lambda i: (i, 0)),
            pl.BlockSpec((BN, 1), lambda i: (i, 0)),
            pl.BlockSpec((BN, 1), lambda i: (i, 0)),
        ],
        out_shape=[
            jax.ShapeDtypeStruct((N, D_OUT), F32),
            jax.ShapeDtypeStruct((N, D_OUT), F32),
            jax.ShapeDtypeStruct((N, 1), F32),
            jax.ShapeDtypeStruct((N, 1), F32),
        ],
    )(x, W, a)


def _tc_edge_maps(ett_p, W, a, E_pad, D_IN, D_REL, D_OUT):
    BE = 8192 if E_pad % 8192 == 0 else E_pad
    DW = W.shape[0]

    def body(ett_ref, W_ref, a_ref, z_ref, s3_ref):
        z = jnp.dot(ett_ref[...], W_ref[2 * D_IN:, :], preferred_element_type=F32)
        z_ref[...] = z
        s3_ref[...] = jnp.dot(z, a_ref[...], preferred_element_type=F32)

    return pl.pallas_call(
        body,
        grid=(E_pad // BE,),
        in_specs=[
            pl.BlockSpec((BE, D_REL), lambda i: (i, 0)),
            pl.BlockSpec((DW, D_OUT), lambda i: (0, 0)),
            pl.BlockSpec((D_OUT, 1), lambda i: (0, 0)),
        ],
        out_specs=[
            pl.BlockSpec((BE, D_OUT), lambda i: (i, 0)),
            pl.BlockSpec((BE, 1), lambda i: (i, 0)),
        ],
        out_shape=[
            jax.ShapeDtypeStruct((E_pad, D_OUT), F32),
            jax.ShapeDtypeStruct((E_pad, 1), F32),
        ],
    )(ett_p, W, a)


def _tc_finalize(h1, denp, acc, N, D_OUT, NC):
    BN = 1000 if N % 1000 == 0 else N

    def body(h1_ref, dp_ref, acc_ref, o_ref):
        den = dp_ref[0] + dp_ref[1]
        r = 1.0 / (den + EPS)
        t = (h1_ref[...] * den + acc_ref[0] + acc_ref[1]) * r
        o_ref[...] = jnp.where(t > 0.0, t, jnp.exp(t) - 1.0)

    return pl.pallas_call(
        body,
        grid=(N // BN,),
        in_specs=[
            pl.BlockSpec((BN, D_OUT), lambda i: (i, 0)),
            pl.BlockSpec((NC, BN, 1), lambda i: (0, i, 0)),
            pl.BlockSpec((NC, BN, D_OUT), lambda i: (0, i, 0)),
        ],
        out_specs=pl.BlockSpec((BN, D_OUT), lambda i: (i, 0)),
        out_shape=jax.ShapeDtypeStruct((N, D_OUT), F32),
    )(h1, denp, acc)


def kernel(edge_index, x, edge_type_embed, W, a):
    N, D_IN = x.shape
    E = edge_index.shape[1]
    D_REL = edge_type_embed.shape[1]
    D_OUT = W.shape[1]

    info = plsc.get_sparse_core_info()
    NC, NS, L = info.num_cores, info.num_subcores, info.num_lanes
    NW = NC * NS

    CHUNK = 2048
    GROUPS = CHUNK // L
    PT = ((E + NW * CHUNK - 1) // (NW * CHUNK)) * CHUNK
    E_pad = PT * NW
    n_chunks = PT // CHUNK
    NB = PT // 128
    NSTR = (N + 1 + 127) // 128
    NT = NSTR * 128
    KSTR = (NSTR + NS - 1) // NS

    mesh = plsc.VectorSubcoreMesh(core_axis_name="c", subcore_axis_name="s")

    src = jnp.concatenate([edge_index[0], jnp.full((E_pad - E,), N, I32)])
    dst = jnp.concatenate([edge_index[1], jnp.zeros((E_pad - E,), I32)])
    ett_p = jnp.pad(edge_type_embed, ((0, E_pad - E), (0, 0)))

    h1, h2, s1, s2 = _tc_node_matmul(x, W, a, N, D_IN, D_OUT)
    z, s3 = _tc_edge_maps(ett_p, W, a, E_pad, D_IN, D_REL, D_OUT)
    s1p = jnp.pad(s1[:, 0], (0, NT - N))
    s2p = jnp.pad(s2[:, 0], (0, NT - N))
    s3p = s3[:, 0]

    CH0 = n_chunks * 2 - (n_chunks * 2) // 3
    CH1 = n_chunks * 2 - CH0
    PT0 = CH0 * CHUNK
    PT1 = CH1 * CHUNK

    def _tile_span(c, s):
        base = jnp.where(c == 0, s * PT0, NS * PT0 + s * PT1)
        nch = jnp.where(c == 0, CH0, CH1)
        return base, nch

    @functools.partial(
        pl.kernel,
        out_type=(jax.ShapeDtypeStruct((E_pad,), F32),
                  jax.ShapeDtypeStruct((NC, NT), F32)),
        mesh=mesh,
        compiler_params=pltpu.CompilerParams(needs_layout_passes=False),
        scratch_types=[
            pltpu.VMEM((NT,), F32),
            pltpu.VMEM((NT,), F32),
            pltpu.VMEM((CHUNK,), I32),
            pltpu.VMEM((CHUNK,), I32),
            pltpu.VMEM((CHUNK,), F32),
            pltpu.VMEM((CHUNK,), F32),
            pltpu.VMEM((128,), I32),
            pltpu.VMEM((128,), F32),
            pltpu.VMEM_SHARED((NT,), F32),
        ],
    )
    def sc_front(src_h, dst_h, s3_h, s1_h, s2_h,
                 p_h, den_h,
                 s1_t, s2_t, srcb, dstb, s3b, pb, isrc, zbuf, den_sp):
        c = lax.axis_index("c")
        s = lax.axis_index("s")
        base, nch = _tile_span(c, s)
        pltpu.sync_copy(s1_h, s1_t)
        pltpu.sync_copy(s2_h, s2_t)

        def z1b(g, _):
            zbuf[pl.ds(g * L, L)] = jnp.zeros((L,), F32)
            return 0

        lax.fori_loop(0, 128 // L, z1b, 0)

        def z2(k, _):
            j = k * NS + s

            @pl.when(j < NSTR)
            def _():
                pltpu.sync_copy(zbuf, den_sp.at[pl.ds(j * 128, 128)])
            return 0

        lax.fori_loop(0, KSTR, z2, 0)
        plsc.subcore_barrier()

        def chunk_body(k, _):
            off = base + k * CHUNK
            pltpu.sync_copy(src_h.at[pl.ds(off, CHUNK)], srcb)
            pltpu.sync_copy(dst_h.at[pl.ds(off, CHUNK)], dstb)
            pltpu.sync_copy(s3_h.at[pl.ds(off, CHUNK)], s3b)

            def g_body(g, _):
                sl = pl.ds(g * L, L)
                e = (plsc.load_gather(s1_t, [srcb[sl]])
                     + plsc.load_gather(s2_t, [dstb[sl]])
                     + s3b[sl])
                e = jnp.where(e > 0.0, e, ALPHA * e)
                pb[sl] = jnp.exp(jnp.minimum(e, ECLAMP))
                return 0

            lax.fori_loop(0, GROUPS, g_body, 0)
            pltpu.sync_copy(pb, p_h.at[pl.ds(off, CHUNK)])

            def blk(j, _):
                def cp(g2, _):
                    isrc[pl.ds(g2 * L, L)] = srcb[pl.ds(j * 128 + g2 * L, L)]
                    return 0

                lax.fori_loop(0, 128 // L, cp, 0)
                pltpu.sync_copy(pb.at[pl.ds(j * 128, 128)], den_sp.at[isrc],
                                add=True)
                return 0

            lax.fori_loop(0, CHUNK // 128, blk, 0)
            return 0

        lax.fori_loop(0, nch, chunk_body, 0)
        plsc.subcore_barrier()

        def outs(k, _):
            j = k * NS + s

            @pl.when(j < NSTR)
            def _():
                pltpu.sync_copy(den_sp.at[pl.ds(j * 128, 128)],
                                den_h.at[c, pl.ds(j * 128, 128)])
            return 0

        lax.fori_loop(0, KSTR, outs, 0)

    p, denp = sc_front(src, dst, s3p, s1p, s2p)

    @functools.partial(
        pl.kernel,
        out_type=jax.ShapeDtypeStruct((NC, NT, D_OUT), F32),
        mesh=mesh,
        compiler_params=pltpu.CompilerParams(needs_layout_passes=False),
        scratch_types=[
            pltpu.VMEM((CHUNK,), I32),
            pltpu.VMEM((CHUNK,), I32),
            pltpu.VMEM((CHUNK,), F32),
            pltpu.VMEM((128,), I32),
            pltpu.VMEM((128,), I32),
            pltpu.VMEM((128, 128), F32),
            pltpu.VMEM((128, 128), F32),
            pltpu.SemaphoreType.DMA,
            pltpu.VMEM_SHARED((NT, 128), F32),
        ],
    )
    def sc_acc(src_h, dst_h, p_h, h2_h, z_h, acc_h,
               srcb, dstb, pb, isrc, idst, rows, zrows, sem, acc_sp):
        c = lax.axis_index("c")
        s = lax.axis_index("s")
        base, nch = _tile_span(c, s)

        def z1(i, _):
            for g in range(128 // L):
                rows[i, pl.ds(g * L, L)] = jnp.zeros((L,), F32)
            return 0

        lax.fori_loop(0, 128, z1, 0)

        def z2(k, _):
            j = k * NS + s

            @pl.when(j < NSTR)
            def _():
                pltpu.sync_copy(rows, acc_sp.at[pl.ds(j * 128, 128), :])
            return 0

        lax.fori_loop(0, KSTR, z2, 0)
        plsc.subcore_barrier()

        def chunk_body(k, _):
            off = base + k * CHUNK
            pltpu.sync_copy(src_h.at[pl.ds(off, CHUNK)], srcb)
            pltpu.sync_copy(dst_h.at[pl.ds(off, CHUNK)], dstb)
            pltpu.sync_copy(p_h.at[pl.ds(off, CHUNK)], pb)

            def blk(j, _):
                def cp(g2, _):
                    sl = pl.ds(g2 * L, L)
                    bsl = pl.ds(j * 128 + g2 * L, L)
                    isrc[sl] = srcb[bsl]
                    idst[sl] = dstb[bsl]
                    return 0

                lax.fori_loop(0, 128 // L, cp, 0)
                gth = pltpu.async_copy(h2_h.at[idst], rows, sem)
                pltpu.sync_copy(z_h.at[pl.ds(off + j * 128, 128), :], zrows)
                gth.wait()

                def se(e, _):
                    idxv = jnp.full((L,), 0, I32) + (j * 128 + e)
                    pv = plsc.load_gather(pb, [idxv])
                    for g in range(128 // L):
                        sl2 = pl.ds(g * L, L)
                        rows[e, sl2] = (rows[e, sl2] + zrows[e, sl2]) * pv
                    return 0

                lax.fori_loop(0, 128, se, 0)
                pltpu.sync_copy(rows, acc_sp.at[isrc], add=True)
                return 0

            lax.fori_loop(0, CHUNK // 128, blk, 0)
            return 0

        lax.fori_loop(0, nch, chunk_body, 0)
        plsc.subcore_barrier()

        def outs(k, _):
            j = k * NS + s

            @pl.when(j < NSTR)
            def _():
                pltpu.sync_copy(acc_sp.at[pl.ds(j * 128, 128), :],
                                acc_h.at[c, pl.ds(j * 128, 128), :])
            return 0

        lax.fori_loop(0, KSTR, outs, 0)

    acc = sc_acc(src, dst, p, h2, z)

    return _tc_finalize(h1, denp.reshape(NC, NT, 1)[:, :N], acc[:, :N],
                        N, D_OUT, NC)


# speedup vs baseline: 1.3238x; 1.0678x over previous
import functools

import jax
import jax.numpy as jnp
from jax import lax
from jax.experimental import pallas as pl
from jax.experimental.pallas import tpu as pltpu
from jax.experimental.pallas import tpu_sc as plsc

ALPHA = 0.2
EPS = 1e-16
ECLAMP = 50.0
F32 = jnp.float32
I32 = jnp.int32


def _tc_node_matmul(x, W, a, N, D_IN, D_OUT):
    BN = 1000 if N % 1000 == 0 else N

    def body(x_ref, W_ref, a_ref, h1_ref, h2_ref, s1_ref, s2_ref):
        xb = x_ref[...]
        h1 = jnp.dot(xb, W_ref[0:D_IN, :], preferred_element_type=F32)
        h2 = jnp.dot(xb, W_ref[D_IN:2 * D_IN, :], preferred_element_type=F32)
        h1_ref[...] = h1
        h2_ref[...] = h2
        av = a_ref[...]
        s1_ref[...] = jnp.dot(h1, av, preferred_element_type=F32)
        s2_ref[...] = jnp.dot(h2, av, preferred_element_type=F32)

    DW = W.shape[0]
    return pl.pallas_call(
        body,
        grid=(N // BN,),
        in_specs=[
            pl.BlockSpec((BN, D_IN), lambda i: (i, 0)),
            pl.BlockSpec((DW, D_OUT), lambda i: (0, 0)),
            pl.BlockSpec((D_OUT, 1), lambda i: (0, 0)),
        ],
        out_specs=[
            pl.BlockSpec((BN, D_OUT), lambda i: (i, 0)),
            pl.BlockSpec((BN, D_OUT), lambda i: (i, 0)),
            pl.BlockSpec((BN, 1), lambda i: (i, 0)),
            pl.BlockSpec((BN, 1), lambda i: (i, 0)),
        ],
        out_shape=[
            jax.ShapeDtypeStruct((N, D_OUT), F32),
            jax.ShapeDtypeStruct((N, D_OUT), F32),
            jax.ShapeDtypeStruct((N, 1), F32),
            jax.ShapeDtypeStruct((N, 1), F32),
        ],
    )(x, W, a)


def _tc_edge_maps(ett_p, W, a, E_pad, D_IN, D_REL, D_OUT):
    BE = 8192 if E_pad % 8192 == 0 else E_pad
    DW = W.shape[0]

    def body(ett_ref, W_ref, a_ref, z_ref, s3_ref):
        z = jnp.dot(ett_ref[...], W_ref[2 * D_IN:, :], preferred_element_type=F32)
        z_ref[...] = z
        s3_ref[...] = jnp.dot(z, a_ref[...], preferred_element_type=F32)

    return pl.pallas_call(
        body,
        grid=(E_pad // BE,),
        in_specs=[
            pl.BlockSpec((BE, D_REL), lambda i: (i, 0)),
            pl.BlockSpec((DW, D_OUT), lambda i: (0, 0)),
            pl.BlockSpec((D_OUT, 1), lambda i: (0, 0)),
        ],
        out_specs=[
            pl.BlockSpec((BE, D_OUT), lambda i: (i, 0)),
            pl.BlockSpec((BE, 1), lambda i: (i, 0)),
        ],
        out_shape=[
            jax.ShapeDtypeStruct((E_pad, D_OUT), F32),
            jax.ShapeDtypeStruct((E_pad, 1), F32),
        ],
    )(ett_p, W, a)


def _tc_finalize(h1, denp, acc, N, D_OUT, NC):
    BN = 1000 if N % 1000 == 0 else N

    def body(h1_ref, dp_ref, acc_ref, o_ref):
        den = dp_ref[0] + dp_ref[1]
        r = 1.0 / (den + EPS)
        t = (h1_ref[...] * den + acc_ref[0] + acc_ref[1]) * r
        o_ref[...] = jnp.where(t > 0.0, t, jnp.exp(t) - 1.0)

    return pl.pallas_call(
        body,
        grid=(N // BN,),
        in_specs=[
            pl.BlockSpec((BN, D_OUT), lambda i: (i, 0)),
            pl.BlockSpec((NC, BN, 1), lambda i: (0, i, 0)),
            pl.BlockSpec((NC, BN, D_OUT), lambda i: (0, i, 0)),
        ],
        out_specs=pl.BlockSpec((BN, D_OUT), lambda i: (i, 0)),
        out_shape=jax.ShapeDtypeStruct((N, D_OUT), F32),
    )(h1, denp, acc)


def kernel(edge_index, x, edge_type_embed, W, a):
    N, D_IN = x.shape
    E = edge_index.shape[1]
    D_REL = edge_type_embed.shape[1]
    D_OUT = W.shape[1]

    info = plsc.get_sparse_core_info()
    NC, NS, L = info.num_cores, info.num_subcores, info.num_lanes
    NW = NC * NS

    CHUNK = 2048
    GROUPS = CHUNK // L
    PT = ((E + NW * CHUNK - 1) // (NW * CHUNK)) * CHUNK
    E_pad = PT * NW
    n_chunks = PT // CHUNK
    NB = PT // 128
    NSTR = (N + 1 + 127) // 128
    NT = NSTR * 128
    KSTR = (NSTR + NS - 1) // NS

    mesh = plsc.VectorSubcoreMesh(core_axis_name="c", subcore_axis_name="s")

    src = jnp.concatenate([edge_index[0], jnp.full((E_pad - E,), N, I32)])
    dst = jnp.concatenate([edge_index[1], jnp.zeros((E_pad - E,), I32)])
    ett_p = jnp.pad(edge_type_embed, ((0, E_pad - E), (0, 0)))

    h1, h2, s1, s2 = _tc_node_matmul(x, W, a, N, D_IN, D_OUT)
    z, s3 = _tc_edge_maps(ett_p, W, a, E_pad, D_IN, D_REL, D_OUT)
    s1p = jnp.pad(s1[:, 0], (0, NT - N))
    s2p = jnp.pad(s2[:, 0], (0, NT - N))
    s3p = s3[:, 0]

    CH0 = n_chunks * 2 - (n_chunks * 2) // 3
    CH1 = n_chunks * 2 - CH0
    PT0 = CH0 * CHUNK
    PT1 = CH1 * CHUNK

    def _tile_span(c, s):
        base = jnp.where(c == 0, s * PT0, NS * PT0 + s * PT1)
        nch = jnp.where(c == 0, CH0, CH1)
        return base, nch

    @functools.partial(
        pl.kernel,
        out_type=(jax.ShapeDtypeStruct((E_pad,), F32),
                  jax.ShapeDtypeStruct((NC, NT), F32)),
        mesh=mesh,
        compiler_params=pltpu.CompilerParams(needs_layout_passes=False),
        scratch_types=[
            pltpu.VMEM((NT,), F32),
            pltpu.VMEM((NT,), F32),
            pltpu.VMEM((CHUNK,), I32),
            pltpu.VMEM((CHUNK,), I32),
            pltpu.VMEM((CHUNK,), F32),
            pltpu.VMEM((CHUNK,), F32),
            pltpu.VMEM((128,), I32),
            pltpu.VMEM((128,), F32),
            pltpu.VMEM_SHARED((NT,), F32),
        ],
    )
    def sc_front(src_h, dst_h, s3_h, s1_h, s2_h,
                 p_h, den_h,
                 s1_t, s2_t, srcb, dstb, s3b, pb, isrc, zbuf, den_sp):
        c = lax.axis_index("c")
        s = lax.axis_index("s")
        base, nch = _tile_span(c, s)
        pltpu.sync_copy(s1_h, s1_t)
        pltpu.sync_copy(s2_h, s2_t)

        def z1b(g, _):
            zbuf[pl.ds(g * L, L)] = jnp.zeros((L,), F32)
            return 0

        lax.fori_loop(0, 128 // L, z1b, 0)

        def z2(k, _):
            j = k * NS + s

            @pl.when(j < NSTR)
            def _():
                pltpu.sync_copy(zbuf, den_sp.at[pl.ds(j * 128, 128)])
            return 0

        lax.fori_loop(0, KSTR, z2, 0)
        plsc.subcore_barrier()

        def chunk_body(k, _):
            off = base + k * CHUNK
            pltpu.sync_copy(src_h.at[pl.ds(off, CHUNK)], srcb)
            pltpu.sync_copy(dst_h.at[pl.ds(off, CHUNK)], dstb)
            pltpu.sync_copy(s3_h.at[pl.ds(off, CHUNK)], s3b)

            def g_body(g, _):
                sl = pl.ds(g * L, L)
                e = (plsc.load_gather(s1_t, [srcb[sl]])
                     + plsc.load_gather(s2_t, [dstb[sl]])
                     + s3b[sl])
                e = jnp.where(e > 0.0, e, ALPHA * e)
                pb[sl] = jnp.exp(jnp.minimum(e, ECLAMP))
                return 0

            lax.fori_loop(0, GROUPS, g_body, 0)
            pltpu.sync_copy(pb, p_h.at[pl.ds(off, CHUNK)])

            def blk(j, _):
                def cp(g2, _):
                    isrc[pl.ds(g2 * L, L)] = srcb[pl.ds(j * 128 + g2 * L, L)]
                    return 0

                lax.fori_loop(0, 128 // L, cp, 0)
                pltpu.sync_copy(pb.at[pl.ds(j * 128, 128)], den_sp.at[isrc],
                                add=True)
                return 0

            lax.fori_loop(0, CHUNK // 128, blk, 0)
            return 0

        lax.fori_loop(0, nch, chunk_body, 0)
        plsc.subcore_barrier()

        def outs(k, _):
            j = k * NS + s

            @pl.when(j < NSTR)
            def _():
                pltpu.sync_copy(den_sp.at[pl.ds(j * 128, 128)],
                                den_h.at[c, pl.ds(j * 128, 128)])
            return 0

        lax.fori_loop(0, KSTR, outs, 0)

    p, denp = sc_front(src, dst, s3p, s1p, s2p)

    @functools.partial(
        pl.kernel,
        out_type=jax.ShapeDtypeStruct((NC, NT, D_OUT), F32),
        mesh=mesh,
        compiler_params=pltpu.CompilerParams(needs_layout_passes=False),
        scratch_types=[
            pltpu.VMEM((CHUNK,), I32),
            pltpu.VMEM((CHUNK,), I32),
            pltpu.VMEM((CHUNK,), F32),
            pltpu.VMEM((2, 64), I32),
            pltpu.VMEM((2, 64), I32),
            pltpu.VMEM((2, 64, 128), F32),
            pltpu.VMEM((2, 64, 128), F32),
            pltpu.SemaphoreType.DMA,
            pltpu.SemaphoreType.DMA,
            pltpu.SemaphoreType.DMA,
            pltpu.SemaphoreType.DMA,
            pltpu.VMEM_SHARED((NT, 128), F32),
        ],
    )
    def sc_acc(src_h, dst_h, p_h, h2_h, z_h, acc_h,
               srcb, dstb, pb, isrc2, idst2, rows2, zrows2,
               semg0, semg1, sems0, sems1, acc_sp):
        c = lax.axis_index("c")
        s = lax.axis_index("s")
        base, nch = _tile_span(c, s)
        semg = (semg0, semg1)
        sems = (sems0, sems1)
        NBK = CHUNK // 64

        def z1(i, _):
            for g in range(128 // L):
                rows2[0, i, pl.ds(g * L, L)] = jnp.zeros((L,), F32)
                rows2[1, i, pl.ds(g * L, L)] = jnp.zeros((L,), F32)
            return 0

        lax.fori_loop(0, 64, z1, 0)

        def z2(k, _):
            j = k * NS + s

            @pl.when(j < NSTR)
            def _():
                pltpu.sync_copy(rows2.at[0], acc_sp.at[pl.ds(j * 128, 64), :])
                pltpu.sync_copy(rows2.at[1], acc_sp.at[pl.ds(j * 128 + 64, 64), :])
            return 0

        lax.fori_loop(0, KSTR, z2, 0)
        plsc.subcore_barrier()

        def build(t, b):
            def cp(g2, _):
                sl = pl.ds(g2 * L, L)
                bsl = pl.ds(b * 64 + g2 * L, L)
                isrc2[t, sl] = srcb[bsl]
                idst2[t, sl] = dstb[bsl]
                return 0

            lax.fori_loop(0, 64 // L, cp, 0)

        def issue(t, b, off):
            pltpu.async_copy(h2_h.at[idst2.at[t]], rows2.at[t], semg[t])
            pltpu.async_copy(z_h.at[pl.ds(off + b * 64, 64), :], zrows2.at[t],
                             semg[t])

        def wait_gather(t):
            pltpu.make_async_copy(h2_h.at[idst2.at[t]], rows2.at[t],
                                  semg[t]).wait()
            pltpu.make_async_copy(z_h.at[pl.ds(0, 64), :], zrows2.at[t],
                                  semg[t]).wait()

        def wait_scatter(t):
            pltpu.make_async_copy(rows2.at[t], acc_sp.at[isrc2.at[t]],
                                  sems[t]).wait()

        def chunk_body(k, _):
            off = base + k * CHUNK
            pltpu.sync_copy(src_h.at[pl.ds(off, CHUNK)], srcb)
            pltpu.sync_copy(dst_h.at[pl.ds(off, CHUNK)], dstb)
            pltpu.sync_copy(p_h.at[pl.ds(off, CHUNK)], pb)

            build(0, 0)
            issue(0, 0, off)

            def pair(q, _):
                for t2 in range(2):
                    b = q * 2 + t2
                    nb = b + 1
                    o = 1 - t2

                    @pl.when(nb < NBK)
                    def _():
                        @pl.when(nb >= 2)
                        def _():
                            wait_scatter(o)
                        build(o, nb)
                        issue(o, nb, off)

                    wait_gather(t2)

                    def se(e, _):
                        idxv = jnp.full((L,), 0, I32) + (b * 64 + e)
                        pv = plsc.load_gather(pb, [idxv])
                        for g in range(128 // L):
                            sl2 = pl.ds(g * L, L)
                            rows2[t2, e, sl2] = (rows2[t2, e, sl2]
                                                 + zrows2[t2, e, sl2]) * pv
                        return 0

                    lax.fori_loop(0, 64, se, 0)
                    pltpu.async_copy(rows2.at[t2], acc_sp.at[isrc2.at[t2]],
                                     sems[t2], add=True)
                return 0

            lax.fori_loop(0, NBK // 2, pair, 0)
            wait_scatter(0)
            wait_scatter(1)
            return 0

        lax.fori_loop(0, nch, chunk_body, 0)
        plsc.subcore_barrier()

        def outs(k, _):
            j = k * NS + s

            @pl.when(j < NSTR)
            def _():
                pltpu.sync_copy(acc_sp.at[pl.ds(j * 128, 128), :],
                                acc_h.at[c, pl.ds(j * 128, 128), :])
            return 0

        lax.fori_loop(0, KSTR, outs, 0)

    acc = sc_acc(src, dst, p, h2, z)

    return _tc_finalize(h1, denp.reshape(NC, NT, 1)[:, :N], acc[:, :N],
                        N, D_OUT, NC)
